# Initial kernel scaffold; baseline (speedup 1.0000x reference)
import jax
import jax.numpy as jnp
from jax.experimental import pallas as pl


def kernel(x, domains, W1, b1, W2, b2):
    raise NotImplementedError("write your pallas kernel here")


# speedup vs baseline: 3.1725x; 3.1725x over previous
import functools

import jax
import jax.numpy as jnp
from jax import lax
from jax.experimental import pallas as pl
from jax.experimental.pallas import tpu as pltpu
from jax.experimental.pallas import tpu_sc as plsc

N = 32768
D = 1024
MID = 256
E = 8
SCALE = 1.0

BLK = 1024
P = N + E * BLK
NB = P // BLK
NBPAD = 48
NC, NS = 2, 16
NW = NC * NS
T = N // NW
R = 32
NCH = T // R
LANES = 16

_mesh = plsc.VectorSubcoreMesh(core_axis_name="c", subcore_axis_name="s")


def _wid():
    return lax.axis_index("s") * NC + lax.axis_index("c")


def _histvec(ref):
    lane = lax.iota(jnp.int32, LANES)

    def body(v, hvec):
        d = ref[pl.ds(v * LANES, LANES)]
        for e in range(E):
            c = jnp.sum((d == e).astype(jnp.int32))
            hvec = jnp.where(lane == e, hvec + c, hvec)
        return hvec

    return lax.fori_loop(0, T // LANES, body, jnp.zeros((LANES,), jnp.int32))


@functools.partial(
    pl.kernel,
    out_type=(
        jax.ShapeDtypeStruct((P, D), jnp.float32),
        jax.ShapeDtypeStruct((N,), jnp.int32),
        jax.ShapeDtypeStruct((NBPAD,), jnp.int32),
    ),
    mesh=_mesh,
    compiler_params=pltpu.CompilerParams(needs_layout_passes=False),
    scratch_types=[
        pltpu.VMEM((T,), jnp.int32),
        pltpu.VMEM((T,), jnp.int32),
        pltpu.VMEM((NW, LANES), jnp.int32),
        pltpu.VMEM((LANES,), jnp.int32),
        pltpu.VMEM_SHARED((NW, LANES), jnp.int32),
        pltpu.VMEM((NCH, R), jnp.int32),
        pltpu.VMEM((T,), jnp.int32),
        pltpu.VMEM((3, R, D), jnp.float32),
        pltpu.VMEM((NBPAD,), jnp.int32),
        pltpu.SemaphoreType.DMA,
        pltpu.SemaphoreType.DMA,
        pltpu.SemaphoreType.DMA,
        pltpu.SemaphoreType.DMA,
        pltpu.SemaphoreType.DMA,
        pltpu.SemaphoreType.DMA,
    ],
)
def _dispatch(dom_hbm, x_hbm, xs_hbm, dest_hbm, blk_hbm,
              dom_v, domm_v, hist_v, hstage, hshared, idx2d, dflat, xbuf,
              blkbuf, sl0, sl1, sl2, ss0, ss1, ss2):
    cid = lax.axis_index("c")
    sid = lax.axis_index("s")
    wid = sid * NC + cid
    widm = sid * NC + (1 - cid)
    tok0 = wid * T
    pltpu.sync_copy(dom_hbm.at[pl.ds(tok0, T)], dom_v)
    pltpu.sync_copy(dom_hbm.at[pl.ds(widm * T, T)], domm_v)

    hstage[...] = _histvec(dom_v)
    pltpu.sync_copy(hstage, hshared.at[wid])
    hstage[...] = _histvec(domm_v)
    pltpu.sync_copy(hstage, hshared.at[widm])
    plsc.subcore_barrier()
    pltpu.sync_copy(hshared, hist_v)

    lane = lax.iota(jnp.int32, LANES)
    totals = jnp.zeros((LANES,), jnp.int32)
    prebase = jnp.zeros((LANES,), jnp.int32)
    for t in range(NW):
        h = hist_v[t]
        totals = totals + h
        prebase = prebase + h * (jnp.int32(t) < wid).astype(jnp.int32)
    padded = jnp.bitwise_and(totals + (BLK - 1), jnp.int32(-BLK))
    incl = jnp.cumsum(padded)
    base = incl - padded
    start = base + prebase
    ctrs = tuple(jnp.sum(jnp.where(lane == e, start, 0)) for e in range(E))

    @pl.when(wid == 0)
    def _():
        incl_s = [jnp.sum(jnp.where(lane == e, incl, 0)) for e in range(E)]
        for g in range(NBPAD // LANES):
            bstart = (lane + g * LANES) * BLK
            cnt = jnp.zeros((LANES,), jnp.int32)
            for e in range(E):
                cnt = cnt + (bstart >= incl_s[e]).astype(jnp.int32)
            blkbuf[pl.ds(g * LANES, LANES)] = cnt
        pltpu.sync_copy(blkbuf, blk_hbm)

    sls, sss = (sl0, sl1, sl2), (ss0, ss1, ss2)
    ld = [None, None, None]
    scat = [None, None, None]
    for ci in range(NCH):
        b = ci % 3
        if scat[b] is not None:
            scat[b].wait()
        ld[b] = pltpu.async_copy(x_hbm.at[pl.ds(tok0 + ci * R, R)],
                                 xbuf.at[b], sls[b])

        def vbody(v, c, ci=ci):
            d = dom_v[pl.ds(ci * R + v * LANES, LANES)]
            cl = list(c)
            dest = jnp.zeros((LANES,), jnp.int32)
            for e in range(E):
                m = d == e
                mi = m.astype(jnp.int32)
                pc = jnp.cumsum(mi)
                dest = jnp.where(m, cl[e] + pc - 1, dest)
                cl[e] = cl[e] + jnp.sum(mi)
            idx2d[ci, pl.ds(v * LANES, LANES)] = dest
            dflat[pl.ds(ci * R + v * LANES, LANES)] = dest
            return tuple(cl)

        ctrs = lax.fori_loop(0, R // LANES, vbody, ctrs)
        if ci >= 2:
            pb = (ci - 2) % 3
            ld[pb].wait()
            scat[pb] = pltpu.async_copy(xbuf.at[pb],
                                        xs_hbm.at[idx2d.at[ci - 2]], sss[pb])
    pltpu.sync_copy(dflat, dest_hbm.at[pl.ds(tok0, T)])
    for ci in (NCH - 2, NCH - 1):
        pb = ci % 3
        ld[pb].wait()
        scat[pb] = pltpu.async_copy(xbuf.at[pb], xs_hbm.at[idx2d.at[ci]],
                                    sss[pb])
    for p in scat:
        p.wait()


def _mlp_body(bm_ref, x_ref, w1_ref, b1_ref, w2_ref, b2_ref, o_ref):
    e = bm_ref[pl.program_id(0)]

    @pl.when(e < E)
    def _():
        xb = x_ref[...]
        h = jnp.maximum(
            jnp.dot(xb.astype(jnp.bfloat16), w1_ref[0],
                    preferred_element_type=jnp.float32)
            + b1_ref[jnp.minimum(e, E - 1)][None, :],
            0.0)
        y = (jnp.dot(h.astype(jnp.bfloat16), w2_ref[0],
                     preferred_element_type=jnp.float32)
             + b2_ref[jnp.minimum(e, E - 1)][None, :])
        o_ref[...] = xb + SCALE * y


_mlp = pl.pallas_call(
    _mlp_body,
    grid_spec=pltpu.PrefetchScalarGridSpec(
        num_scalar_prefetch=1,
        grid=(NB,),
        in_specs=[
            pl.BlockSpec((BLK, D), lambda b, bm: (jnp.where(bm[b] == E, 0, b), 0)),
            pl.BlockSpec((1, D, MID),
                         lambda b, bm: (jnp.minimum(bm[b], E - 1), 0, 0)),
            pl.BlockSpec((E, MID), lambda b, bm: (0, 0)),
            pl.BlockSpec((1, MID, D),
                         lambda b, bm: (jnp.minimum(bm[b], E - 1), 0, 0)),
            pl.BlockSpec((E, D), lambda b, bm: (0, 0)),
        ],
        out_specs=pl.BlockSpec((BLK, D),
                               lambda b, bm: (jnp.where(bm[b] == E, NB, b), 0)),
    ),
    out_shape=jax.ShapeDtypeStruct((P + BLK, D), jnp.float32),
)


@functools.partial(
    pl.kernel,
    out_type=jax.ShapeDtypeStruct((N, D), jnp.float32),
    mesh=_mesh,
    compiler_params=pltpu.CompilerParams(needs_layout_passes=False),
    scratch_types=[
        pltpu.VMEM((T,), jnp.int32),
        pltpu.VMEM((3, R, D), jnp.float32),
        pltpu.SemaphoreType.DMA,
        pltpu.SemaphoreType.DMA,
        pltpu.SemaphoreType.DMA,
        pltpu.SemaphoreType.DMA,
        pltpu.SemaphoreType.DMA,
        pltpu.SemaphoreType.DMA,
    ],
)
def _unsort(dest_hbm, os_hbm, out_hbm, dv, rbuf, sg0, sg1, sg2, sw0, sw1, sw2):
    wid = _wid()
    tok0 = wid * T
    pltpu.sync_copy(dest_hbm.at[pl.ds(tok0, T)], dv)
    sgs, sws = (sg0, sg1, sg2), (sw0, sw1, sw2)
    g = [None, None, None]
    wr = [None, None, None]
    for ci in range(NCH):
        b = ci % 3
        if wr[b] is not None:
            wr[b].wait()
        g[b] = pltpu.async_copy(os_hbm.at[dv.at[pl.ds(ci * R, R)]],
                                rbuf.at[b], sgs[b])
        if ci >= 2:
            pb = (ci - 2) % 3
            g[pb].wait()
            wr[pb] = pltpu.async_copy(
                rbuf.at[pb], out_hbm.at[pl.ds(tok0 + (ci - 2) * R, R)],
                sws[pb])
    for ci in (NCH - 2, NCH - 1):
        pb = ci % 3
        g[pb].wait()
        wr[pb] = pltpu.async_copy(rbuf.at[pb],
                                  out_hbm.at[pl.ds(tok0 + ci * R, R)],
                                  sws[pb])
    for p in wr:
        p.wait()


def kernel(x, domains, W1, b1, W2, b2):
    xs, dest, blkmap = _dispatch(domains, x)
    os_ = _mlp(blkmap, xs, W1.astype(jnp.bfloat16), b1,
               W2.astype(jnp.bfloat16), b2)
    return _unsort(dest, os_)
